# baseline (speedup 1.0000x reference)
import jax
import jax.numpy as jnp
from jax.experimental import pallas as pl
from jax.experimental.pallas import tpu as pltpu

_B = 1024
_N = 1024
_IN_DIM = 128
_LAT = 64
_REG = 0.05
_NITER = 1000
_STRIDE = 5


def _novae_body(x_ref, zp_ref,
                ew0, eb0, ew1, eb1, ew2, eb2, ew3, eb3,
                dw0, db0, dw1, db1, dw2, db2, dw3, db3,
                out_ref):
    f32 = jnp.float32

    h = x_ref[...]
    h = jnp.maximum(jnp.dot(h, ew0[...], preferred_element_type=f32) + eb0[...][None, :], 0.0)
    h = jnp.maximum(jnp.dot(h, ew1[...], preferred_element_type=f32) + eb1[...][None, :], 0.0)
    h = jnp.maximum(jnp.dot(h, ew2[...], preferred_element_type=f32) + eb2[...][None, :], 0.0)
    z = jnp.dot(h, ew3[...], preferred_element_type=f32) + eb3[...][None, :]

    zp = zp_ref[...]
    zn = jnp.sum(z * z, axis=1, keepdims=True)
    zpn = jnp.sum(zp * zp, axis=1, keepdims=True)
    cross = jax.lax.dot_general(z, zp, (((1,), (1,)), ((), ())),
                                preferred_element_type=f32)
    sq = zn + zpn.T - 2.0 * cross
    maxm = jnp.maximum(jnp.max(sq), 0.0)
    c = f32(1.0 / _REG) / (maxm + 1e-12)
    k = jnp.exp(jnp.minimum(sq * -c, 0.0))
    a = f32(1.0 / _B)
    bm = f32(1.0 / _N)

    def half_u(v_row):
        return a / (jnp.sum(k * v_row, axis=1, keepdims=True) + 1e-16)

    def half_v(u_col):
        return bm / (jnp.sum(k * u_col, axis=0, keepdims=True) + 1e-16)

    u_col = a / (jnp.sum(k, axis=1, keepdims=True) + 1e-16)
    v_row = half_v(u_col)
    u_col = half_u(v_row)
    v_prev, v_row = v_row, half_v(u_col)
    u_col = half_u(v_row)
    v_prev, v_row = v_row, half_v(u_col)
    done0 = jnp.max(jnp.abs(v_row - v_prev) - f32(1e-5) * v_row) <= 0.0

    def cond(carry):
        it, _, _, done = carry
        return jnp.logical_and(it < _NITER, jnp.logical_not(done))

    def body(carry):
        it, _, v_row, _ = carry
        u_new = half_u(v_row)
        v_new = half_v(u_new)
        done = jnp.max(jnp.abs(v_new - v_row) - f32(1e-5) * v_new) <= 0.0
        return it + 1, u_new, v_new, done

    _, u_col, v_row, _ = jax.lax.while_loop(
        cond, body, (jnp.int32(3), u_col, v_row, done0))

    v_col = v_row.reshape(_N, 1)
    z_sel = u_col * jnp.dot(k, v_col * zp, preferred_element_type=f32)

    h = jnp.maximum(jnp.dot(z_sel, dw0[...], preferred_element_type=f32) + db0[...][None, :], 0.0)
    h = jnp.maximum(jnp.dot(h, dw1[...], preferred_element_type=f32) + db1[...][None, :], 0.0)
    h = jnp.maximum(jnp.dot(h, dw2[...], preferred_element_type=f32) + db2[...][None, :], 0.0)
    out_ref[...] = jnp.dot(h, dw3[...], preferred_element_type=f32) + db3[...][None, :]


def kernel(x, z_prior, enc_W0, enc_b0, enc_W1, enc_b1, enc_W2, enc_b2,
           enc_W3, enc_b3, dec_W0, dec_b0, dec_W1, dec_b1, dec_W2, dec_b2,
           dec_W3, dec_b3):
    return pl.pallas_call(
        _novae_body,
        out_shape=jax.ShapeDtypeStruct((_B, _IN_DIM), jnp.float32),
        compiler_params=pltpu.CompilerParams(
            vmem_limit_bytes=100 * 1024 * 1024),
    )(x, z_prior,
      enc_W0, enc_b0, enc_W1, enc_b1, enc_W2, enc_b2, enc_W3, enc_b3,
      dec_W0, dec_b0, dec_W1, dec_b1, dec_W2, dec_b2, dec_W3, dec_b3)


# speedup vs baseline: 1.0250x; 1.0250x over previous
import jax
import jax.numpy as jnp
from jax.experimental import pallas as pl
from jax.experimental.pallas import tpu as pltpu

_B = 1024
_N = 1024
_IN_DIM = 128
_LAT = 64
_REG = 0.05
_NITER = 1000


def _novae_body(x_ref, zp_ref,
                ew0, eb0, ew1, eb1, ew2, eb2, ew3, eb3,
                dw0_h, db0_h, dw1_h, db1_h, dw2_h, db2_h, dw3_h, db3_h,
                out_ref,
                dw0, db0, dw1, db1, dw2, db2, dw3, db3, sems):
    f32 = jnp.float32

    copies = [
        pltpu.make_async_copy(dw0_h, dw0, sems.at[0]),
        pltpu.make_async_copy(db0_h, db0, sems.at[1]),
        pltpu.make_async_copy(dw1_h, dw1, sems.at[2]),
        pltpu.make_async_copy(db1_h, db1, sems.at[3]),
        pltpu.make_async_copy(dw2_h, dw2, sems.at[4]),
        pltpu.make_async_copy(db2_h, db2, sems.at[5]),
        pltpu.make_async_copy(dw3_h, dw3, sems.at[6]),
        pltpu.make_async_copy(db3_h, db3, sems.at[7]),
    ]
    for cp in copies:
        cp.start()

    h = x_ref[...]
    h = jnp.maximum(jnp.dot(h, ew0[...], preferred_element_type=f32) + eb0[...][None, :], 0.0)
    h = jnp.maximum(jnp.dot(h, ew1[...], preferred_element_type=f32) + eb1[...][None, :], 0.0)
    h = jnp.maximum(jnp.dot(h, ew2[...], preferred_element_type=f32) + eb2[...][None, :], 0.0)
    z = jnp.dot(h, ew3[...], preferred_element_type=f32) + eb3[...][None, :]

    zp = zp_ref[...]
    zn = jnp.sum(z * z, axis=1, keepdims=True)
    zpn = jnp.sum(zp * zp, axis=1, keepdims=True)
    cross = jax.lax.dot_general(z, zp, (((1,), (1,)), ((), ())),
                                preferred_element_type=f32)
    sq = zn + zpn.T - 2.0 * cross
    maxm = jnp.maximum(jnp.max(sq), 0.0)
    c = f32(1.0 / _REG) / (maxm + 1e-12)
    k = jnp.exp(jnp.minimum(sq * -c, 0.0))
    a = f32(1.0 / _B)
    bm = f32(1.0 / _N)

    def half_u(v_row):
        return a / (jnp.sum(k * v_row, axis=1, keepdims=True) + 1e-16)

    def half_v(u_col):
        return bm / (jnp.sum(k * u_col, axis=0, keepdims=True) + 1e-16)

    v_row = bm / (jnp.sum(k, axis=0, keepdims=True) + 1e-16)
    u_col = half_u(v_row)
    v_row = half_v(u_col)
    u_col = half_u(v_row)
    v_prev, v_row = v_row, half_v(u_col)
    done0 = jnp.max(jnp.abs(v_row - v_prev) - f32(1e-5) * v_row) <= 0.0

    def cond(carry):
        it, _, _, done = carry
        return jnp.logical_and(it < _NITER, jnp.logical_not(done))

    def body(carry):
        it, _, v_row, _ = carry
        u_new = half_u(v_row)
        v_new = half_v(u_new)
        done = jnp.max(jnp.abs(v_new - v_row) - f32(1e-5) * v_new) <= 0.0
        return it + 1, u_new, v_new, done

    _, u_col, v_row, _ = jax.lax.while_loop(
        cond, body, (jnp.int32(2), u_col, v_row, done0))

    v_col = v_row.reshape(_N, 1)
    z_sel = u_col * jnp.dot(k, v_col * zp, preferred_element_type=f32)

    for cp in copies:
        cp.wait()
    h = jnp.maximum(jnp.dot(z_sel, dw0[...], preferred_element_type=f32) + db0[...][None, :], 0.0)
    h = jnp.maximum(jnp.dot(h, dw1[...], preferred_element_type=f32) + db1[...][None, :], 0.0)
    h = jnp.maximum(jnp.dot(h, dw2[...], preferred_element_type=f32) + db2[...][None, :], 0.0)
    out_ref[...] = jnp.dot(h, dw3[...], preferred_element_type=f32) + db3[...][None, :]


def kernel(x, z_prior, enc_W0, enc_b0, enc_W1, enc_b1, enc_W2, enc_b2,
           enc_W3, enc_b3, dec_W0, dec_b0, dec_W1, dec_b1, dec_W2, dec_b2,
           dec_W3, dec_b3):
    vmem_spec = pl.BlockSpec(memory_space=pltpu.MemorySpace.VMEM)
    hbm_spec = pl.BlockSpec(memory_space=pltpu.MemorySpace.HBM)
    dec_arrays = (dec_W0, dec_b0, dec_W1, dec_b1, dec_W2, dec_b2, dec_W3, dec_b3)
    return pl.pallas_call(
        _novae_body,
        out_shape=jax.ShapeDtypeStruct((_B, _IN_DIM), jnp.float32),
        in_specs=[vmem_spec] * 10 + [hbm_spec] * 8,
        scratch_shapes=[pltpu.VMEM(arr.shape, arr.dtype) for arr in dec_arrays]
        + [pltpu.SemaphoreType.DMA((8,))],
        compiler_params=pltpu.CompilerParams(
            vmem_limit_bytes=100 * 1024 * 1024),
    )(x, z_prior,
      enc_W0, enc_b0, enc_W1, enc_b1, enc_W2, enc_b2, enc_W3, enc_b3,
      dec_W0, dec_b0, dec_W1, dec_b1, dec_W2, dec_b2, dec_W3, dec_b3)
